# baseline (speedup 1.0000x reference)
import functools

import jax
import jax.numpy as jnp
from jax import lax
from jax.experimental import pallas as pl
from jax.experimental.pallas import tpu as pltpu
from jax.experimental.pallas import tpu_sc as plsc

N = 10000
E = 320000
D = 128
NA = 729
NC = 2
NS = 16
NW = NC * NS
CH = 128
EPW = 10240
NCH = EPW // CH
EPAD = NW * EPW - E
NP = 10240
RPS = NP // NS

BLK = 1000
G = N // BLK


@functools.lru_cache(maxsize=None)
def _sc_kernels():
    mesh = plsc.VectorSubcoreMesh(core_axis_name="c", subcore_axis_name="s",
                                  num_cores=NC, num_subcores=NS)

    @functools.partial(
        pl.kernel,
        out_type=jax.ShapeDtypeStruct((NC, NP, D), jnp.float32),
        mesh=mesh,
        scratch_types=[
            pltpu.VMEM((NCH, CH), jnp.int32),
            pltpu.VMEM((CH, D), jnp.float32),
            pltpu.VMEM_SHARED((NP, D), jnp.float32),
        ],
    )
    def _sc_degree(dst_hbm, ones_hbm, zeros_hbm, out_hbm, dst_v, ones_v, deg_sh):
        cid = lax.axis_index("c")
        sid = lax.axis_index("s")
        wid = cid * NS + sid
        pltpu.sync_copy(zeros_hbm, deg_sh.at[pl.ds(sid * RPS, RPS)])
        pltpu.sync_copy(dst_hbm.at[wid], dst_v)
        pltpu.sync_copy(ones_hbm, ones_v)
        plsc.subcore_barrier()

        def step(j, carry):
            pltpu.sync_copy(ones_v, deg_sh.at[dst_v.at[j]], add=True)
            return carry

        lax.fori_loop(0, NCH, step, 0)
        plsc.subcore_barrier()
        pltpu.sync_copy(deg_sh.at[pl.ds(sid * RPS, RPS)],
                        out_hbm.at[cid, pl.ds(sid * RPS, RPS)])

    @functools.partial(
        pl.kernel,
        out_type=jax.ShapeDtypeStruct((NC, NP, D), jnp.float32),
        mesh=mesh,
        scratch_types=[
            pltpu.VMEM((2, NCH, CH), jnp.int32),
            pltpu.VMEM((CH, D), jnp.float32),
            pltpu.VMEM_SHARED((NP, D), jnp.float32),
            pltpu.SemaphoreType.DMA,
        ],
    )
    def _sc_scatter(t_hbm, idx_hbm, zeros_hbm, out_hbm,
                    idx_v, rows_v, acc_sh, sem):
        cid = lax.axis_index("c")
        sid = lax.axis_index("s")
        wid = cid * NS + sid
        pltpu.sync_copy(zeros_hbm, acc_sh.at[pl.ds(sid * RPS, RPS)])
        pltpu.sync_copy(idx_hbm.at[wid], idx_v)
        plsc.subcore_barrier()

        def step(j, carry):
            pltpu.async_copy(t_hbm.at[idx_v.at[0, j]], rows_v, sem).wait()
            pltpu.sync_copy(rows_v, acc_sh.at[idx_v.at[1, j]], add=True)
            return carry

        lax.fori_loop(0, NCH, step, 0)
        plsc.subcore_barrier()
        pltpu.sync_copy(acc_sh.at[pl.ds(sid * RPS, RPS)],
                        out_hbm.at[cid, pl.ds(sid * RPS, RPS)])

    return _sc_degree, _sc_scatter


def _tc_dis(d0, d1):
    def body(a_ref, b_ref, o_ref):
        o_ref[...] = lax.rsqrt(a_ref[...] + b_ref[...] + 1.0)

    return pl.pallas_call(
        body, out_shape=jax.ShapeDtypeStruct((N,), jnp.float32))(d0, d1)


def _tc_mmscale(x, W, disc):
    def body(x_ref, w_ref, d_ref, o_ref):
        o_ref[...] = jnp.dot(x_ref[...], w_ref[...],
                             preferred_element_type=jnp.float32) * d_ref[...]

    return pl.pallas_call(
        body,
        grid=(G,),
        in_specs=[pl.BlockSpec((BLK, D), lambda i: (i, 0)),
                  pl.BlockSpec((D, D), lambda i: (0, 0)),
                  pl.BlockSpec((BLK, 1), lambda i: (i, 0))],
        out_specs=pl.BlockSpec((BLK, D), lambda i: (i, 0)),
        out_shape=jax.ShapeDtypeStruct((N, D), jnp.float32),
    )(x, W, disc)


def _tc_layer(s0, s1, t, disc, b, W):
    def body(s0_ref, s1_ref, t_ref, d_ref, b_ref, w_ref, o_ref):
        h = jnp.maximum(
            (s0_ref[...] + s1_ref[...] + t_ref[...]) * d_ref[...] + b_ref[...],
            0.0)
        o_ref[...] = jnp.dot(h, w_ref[...],
                             preferred_element_type=jnp.float32) * d_ref[...]

    return pl.pallas_call(
        body,
        grid=(G,),
        in_specs=[pl.BlockSpec((BLK, D), lambda i: (i, 0)),
                  pl.BlockSpec((BLK, D), lambda i: (i, 0)),
                  pl.BlockSpec((BLK, D), lambda i: (i, 0)),
                  pl.BlockSpec((BLK, 1), lambda i: (i, 0)),
                  pl.BlockSpec((1, D), lambda i: (0, 0)),
                  pl.BlockSpec((D, D), lambda i: (0, 0))],
        out_specs=pl.BlockSpec((BLK, D), lambda i: (i, 0)),
        out_shape=jax.ShapeDtypeStruct((N, D), jnp.float32),
    )(s0, s1, t, disc, b, W)


def _tc_head(s0, s1, t, disc, b, Wa, ba, Wc, bc):
    def body(s0_ref, s1_ref, t_ref, d_ref, b_ref, wa_ref, ba_ref, wc_ref,
             bc_ref, lo_ref, vo_ref, acc_ref):
        i = pl.program_id(0)
        h = jnp.maximum(
            (s0_ref[...] + s1_ref[...] + t_ref[...]) * d_ref[...] + b_ref[...],
            0.0)
        ps = jnp.sum(h, axis=0, keepdims=True)

        @pl.when(i == 0)
        def _():
            acc_ref[...] = ps

        @pl.when(i > 0)
        def _():
            acc_ref[...] += ps

        @pl.when(i == G - 1)
        def _():
            gf = acc_ref[...] * (1.0 / N)
            lo_ref[...] = jnp.dot(gf, wa_ref[...],
                                  preferred_element_type=jnp.float32) + ba_ref[...]
            vo_ref[...] = jnp.dot(gf, wc_ref[...],
                                  preferred_element_type=jnp.float32) + bc_ref[...]

    return pl.pallas_call(
        body,
        grid=(G,),
        in_specs=[pl.BlockSpec((BLK, D), lambda i: (i, 0)),
                  pl.BlockSpec((BLK, D), lambda i: (i, 0)),
                  pl.BlockSpec((BLK, D), lambda i: (i, 0)),
                  pl.BlockSpec((BLK, 1), lambda i: (i, 0)),
                  pl.BlockSpec((1, D), lambda i: (0, 0)),
                  pl.BlockSpec((D, NA), lambda i: (0, 0)),
                  pl.BlockSpec((1, NA), lambda i: (0, 0)),
                  pl.BlockSpec((D, 1), lambda i: (0, 0)),
                  pl.BlockSpec((1, 1), lambda i: (0, 0))],
        out_specs=[pl.BlockSpec((1, NA), lambda i: (0, 0)),
                   pl.BlockSpec((1, 1), lambda i: (0, 0))],
        out_shape=[jax.ShapeDtypeStruct((1, NA), jnp.float32),
                   jax.ShapeDtypeStruct((1, 1), jnp.float32)],
        scratch_shapes=[pltpu.VMEM((1, D), jnp.float32)],
    )(s0, s1, t, disc, b, Wa, ba, Wc, bc)


def kernel(x, edge_index, W1, b1, W2, b2, W3, b3, Wa, ba, Wc, bc):
    ei = edge_index.astype(jnp.int32)
    srcp = jnp.concatenate([ei[0], jnp.zeros((EPAD,), jnp.int32)])
    dstp = jnp.concatenate([ei[1], jnp.full((EPAD,), N, jnp.int32)])
    src3 = srcp.reshape(NW, NCH, CH)
    dst3 = dstp.reshape(NW, NCH, CH)
    idx3 = jnp.stack([src3, dst3], axis=1)

    onesd = jnp.ones((CH, D), jnp.float32)
    zrows = jnp.zeros((RPS, D), jnp.float32)

    _sc_degree, _sc_scatter = _sc_kernels()
    degp = _sc_degree(dst3, onesd, zrows)
    dis = _tc_dis(degp[0, :N, 0], degp[1, :N, 0])
    disc = dis.reshape(N, 1)

    b1r, b2r, b3r = b1.reshape(1, D), b2.reshape(1, D), b3.reshape(1, D)

    t = _tc_mmscale(x, W1, disc)
    sp = _sc_scatter(t, idx3, zrows)
    t = _tc_layer(sp[0], sp[1], t, disc, b1r, W2)
    sp = _sc_scatter(t, idx3, zrows)
    t = _tc_layer(sp[0], sp[1], t, disc, b2r, W3)
    sp = _sc_scatter(t, idx3, zrows)
    logits, value = _tc_head(sp[0], sp[1], t, disc, b3r,
                             Wa, ba.reshape(1, NA), Wc, bc.reshape(1, 1))
    return logits.reshape(NA), value.reshape(())


# speedup vs baseline: 2.3775x; 2.3775x over previous
import functools

import jax
import jax.numpy as jnp
from jax import lax
from jax.experimental import pallas as pl
from jax.experimental.pallas import tpu as pltpu
from jax.experimental.pallas import tpu_sc as plsc

N = 10000
E = 320000
D = 128
NA = 729
NC = 2
NS = 16
NW = NC * NS
EPW = E // NW
CH = 125
NCH = EPW // CH
NP = 10240
RPS = NP // NS

BLK = 1000
G = N // BLK


@functools.lru_cache(maxsize=None)
def _sc_kernels():
    mesh = plsc.VectorSubcoreMesh(core_axis_name="c", subcore_axis_name="s",
                                  num_cores=NC, num_subcores=NS)

    @functools.partial(
        pl.kernel,
        out_type=jax.ShapeDtypeStruct((NC, NP, D), jnp.float32),
        mesh=mesh,
        scratch_types=[
            pltpu.VMEM((NCH, CH), jnp.int32),
            pltpu.VMEM((CH, D), jnp.float32),
            pltpu.VMEM_SHARED((NP, D), jnp.float32),
        ],
    )
    def _sc_degree(dst_hbm, ones_hbm, zeros_hbm, out_hbm, dst_v, ones_v, deg_sh):
        cid = lax.axis_index("c")
        sid = lax.axis_index("s")
        wid = cid * NS + sid
        pltpu.sync_copy(zeros_hbm, deg_sh.at[pl.ds(sid * RPS, RPS)])
        pltpu.sync_copy(dst_hbm.at[wid], dst_v)
        pltpu.sync_copy(ones_hbm, ones_v)
        plsc.subcore_barrier()

        def step(j, carry):
            pltpu.sync_copy(ones_v, deg_sh.at[dst_v.at[j]], add=True)
            return carry

        lax.fori_loop(0, NCH, step, 0)
        plsc.subcore_barrier()
        pltpu.sync_copy(deg_sh.at[pl.ds(sid * RPS, RPS)],
                        out_hbm.at[cid, pl.ds(sid * RPS, RPS)])

    @functools.partial(
        pl.kernel,
        out_type=jax.ShapeDtypeStruct((NC, NP, D), jnp.float32),
        mesh=mesh,
        scratch_types=[
            pltpu.VMEM((2, NCH, CH), jnp.int32),
            pltpu.VMEM((CH, D), jnp.float32),
            pltpu.VMEM_SHARED((NP, D), jnp.float32),
            pltpu.SemaphoreType.DMA,
        ],
    )
    def _sc_scatter(t_hbm, idx_hbm, zeros_hbm, out_hbm,
                    idx_v, rows_v, acc_sh, sem):
        cid = lax.axis_index("c")
        sid = lax.axis_index("s")
        wid = cid * NS + sid
        pltpu.sync_copy(zeros_hbm, acc_sh.at[pl.ds(sid * RPS, RPS)])
        pltpu.sync_copy(idx_hbm.at[wid], idx_v)
        plsc.subcore_barrier()

        def step(j, carry):
            pltpu.async_copy(t_hbm.at[idx_v.at[0, j]], rows_v, sem).wait()
            pltpu.sync_copy(rows_v, acc_sh.at[idx_v.at[1, j]], add=True)
            return carry

        lax.fori_loop(0, NCH, step, 0)
        plsc.subcore_barrier()
        pltpu.sync_copy(acc_sh.at[pl.ds(sid * RPS, RPS)],
                        out_hbm.at[cid, pl.ds(sid * RPS, RPS)])

    return _sc_degree, _sc_scatter


def _tc_dis(d0, d1):
    def body(a_ref, b_ref, o_ref):
        o_ref[...] = lax.rsqrt(a_ref[...] + b_ref[...] + 1.0)

    return pl.pallas_call(
        body, out_shape=jax.ShapeDtypeStruct((N,), jnp.float32))(d0, d1)


def _tc_mmscale(x, W, disc):
    def body(x_ref, w_ref, d_ref, o_ref):
        o_ref[...] = jnp.dot(x_ref[...], w_ref[...],
                             preferred_element_type=jnp.float32) * d_ref[...]

    return pl.pallas_call(
        body,
        grid=(G,),
        in_specs=[pl.BlockSpec((BLK, D), lambda i: (i, 0)),
                  pl.BlockSpec((D, D), lambda i: (0, 0)),
                  pl.BlockSpec((BLK, 1), lambda i: (i, 0))],
        out_specs=pl.BlockSpec((BLK, D), lambda i: (i, 0)),
        out_shape=jax.ShapeDtypeStruct((N, D), jnp.float32),
    )(x, W, disc)


def _tc_layer(s0, s1, t, disc, b, W):
    def body(s0_ref, s1_ref, t_ref, d_ref, b_ref, w_ref, o_ref):
        h = jnp.maximum(
            (s0_ref[...] + s1_ref[...] + t_ref[...]) * d_ref[...] + b_ref[...],
            0.0)
        o_ref[...] = jnp.dot(h, w_ref[...],
                             preferred_element_type=jnp.float32) * d_ref[...]

    return pl.pallas_call(
        body,
        grid=(G,),
        in_specs=[pl.BlockSpec((BLK, D), lambda i: (i, 0)),
                  pl.BlockSpec((BLK, D), lambda i: (i, 0)),
                  pl.BlockSpec((BLK, D), lambda i: (i, 0)),
                  pl.BlockSpec((BLK, 1), lambda i: (i, 0)),
                  pl.BlockSpec((1, D), lambda i: (0, 0)),
                  pl.BlockSpec((D, D), lambda i: (0, 0))],
        out_specs=pl.BlockSpec((BLK, D), lambda i: (i, 0)),
        out_shape=jax.ShapeDtypeStruct((N, D), jnp.float32),
    )(s0, s1, t, disc, b, W)


def _tc_head(s0, s1, t, disc, b, Wa, ba, Wc, bc):
    def body(s0_ref, s1_ref, t_ref, d_ref, b_ref, wa_ref, ba_ref, wc_ref,
             bc_ref, lo_ref, vo_ref, acc_ref):
        i = pl.program_id(0)
        h = jnp.maximum(
            (s0_ref[...] + s1_ref[...] + t_ref[...]) * d_ref[...] + b_ref[...],
            0.0)
        ps = jnp.sum(h, axis=0, keepdims=True)

        @pl.when(i == 0)
        def _():
            acc_ref[...] = ps

        @pl.when(i > 0)
        def _():
            acc_ref[...] += ps

        @pl.when(i == G - 1)
        def _():
            gf = acc_ref[...] * (1.0 / N)
            lo_ref[...] = jnp.dot(gf, wa_ref[...],
                                  preferred_element_type=jnp.float32) + ba_ref[...]
            vo_ref[...] = jnp.dot(gf, wc_ref[...],
                                  preferred_element_type=jnp.float32) + bc_ref[...]

    return pl.pallas_call(
        body,
        grid=(G,),
        in_specs=[pl.BlockSpec((BLK, D), lambda i: (i, 0)),
                  pl.BlockSpec((BLK, D), lambda i: (i, 0)),
                  pl.BlockSpec((BLK, D), lambda i: (i, 0)),
                  pl.BlockSpec((BLK, 1), lambda i: (i, 0)),
                  pl.BlockSpec((1, D), lambda i: (0, 0)),
                  pl.BlockSpec((D, NA), lambda i: (0, 0)),
                  pl.BlockSpec((1, NA), lambda i: (0, 0)),
                  pl.BlockSpec((D, 1), lambda i: (0, 0)),
                  pl.BlockSpec((1, 1), lambda i: (0, 0))],
        out_specs=[pl.BlockSpec((1, NA), lambda i: (0, 0)),
                   pl.BlockSpec((1, 1), lambda i: (0, 0))],
        out_shape=[jax.ShapeDtypeStruct((1, NA), jnp.float32),
                   jax.ShapeDtypeStruct((1, 1), jnp.float32)],
        scratch_shapes=[pltpu.VMEM((1, D), jnp.float32)],
    )(s0, s1, t, disc, b, Wa, ba, Wc, bc)


def kernel(x, edge_index, W1, b1, W2, b2, W3, b3, Wa, ba, Wc, bc):
    ei = edge_index.astype(jnp.int32)
    src3 = ei[0].reshape(NW, NCH, CH)
    dst3 = ei[1].reshape(NW, NCH, CH)
    idx3 = jnp.stack([src3, dst3], axis=1)

    onesd = jnp.ones((CH, D), jnp.float32)
    zrows = jnp.zeros((RPS, D), jnp.float32)

    _sc_degree, _sc_scatter = _sc_kernels()
    degp = _sc_degree(dst3, onesd, zrows)
    dis = _tc_dis(degp[0, :N, 0], degp[1, :N, 0])
    disc = dis.reshape(N, 1)

    b1r, b2r, b3r = b1.reshape(1, D), b2.reshape(1, D), b3.reshape(1, D)

    t = _tc_mmscale(x, W1, disc)
    sp = _sc_scatter(t, idx3, zrows)
    t = _tc_layer(sp[0], sp[1], t, disc, b1r, W2)
    sp = _sc_scatter(t, idx3, zrows)
    t = _tc_layer(sp[0], sp[1], t, disc, b2r, W3)
    sp = _sc_scatter(t, idx3, zrows)
    logits, value = _tc_head(sp[0], sp[1], t, disc, b3r,
                             Wa, ba.reshape(1, NA), Wc, bc.reshape(1, 1))
    return logits.reshape(NA), value.reshape(())
